# baseline (speedup 1.0000x reference)
import functools

import jax
import jax.numpy as jnp
from jax.experimental import pallas as pl

B, P, N, C, H = 16, 512, 20, 9, 256
NEG = -1000000000.0
RPL = 256


def _mlp_pool_kernel(x_ref, m_ref, w1_ref, b1_ref, w2_ref, b2_ref, o_ref):
    x = x_ref[...].reshape(N * RPL, C)
    h1 = jnp.maximum(
        jnp.dot(x, w1_ref[...], preferred_element_type=jnp.float32) + b1_ref[...],
        0.0,
    )
    h2 = jnp.dot(h1, w2_ref[...], preferred_element_type=jnp.float32) + b2_ref[...]
    h3 = h2.reshape(N, RPL, H)
    acc = jnp.full((RPL, H), NEG, dtype=jnp.float32)
    for i in range(N):
        acc = jnp.maximum(acc, jnp.where(m_ref[i][:, None] > 0, h3[i], NEG))
    o_ref[...] = jnp.where(acc == NEG, 0.0, acc)


@jax.jit
def kernel(polylines, polylines_mask, W1, b1, W2, b2):
    BP = B * P
    x = polylines.reshape(BP, N, C).transpose(1, 0, 2)
    m = polylines_mask.reshape(BP, N).T.astype(jnp.float32)
    b1r = b1.reshape(1, H)
    b2r = b2.reshape(1, H)
    grid = BP // RPL
    out = pl.pallas_call(
        _mlp_pool_kernel,
        grid=(grid,),
        in_specs=[
            pl.BlockSpec((N, RPL, C), lambda g: (0, g, 0)),
            pl.BlockSpec((N, RPL), lambda g: (0, g)),
            pl.BlockSpec((C, H), lambda g: (0, 0)),
            pl.BlockSpec((1, H), lambda g: (0, 0)),
            pl.BlockSpec((H, H), lambda g: (0, 0)),
            pl.BlockSpec((1, H), lambda g: (0, 0)),
        ],
        out_specs=pl.BlockSpec((RPL, H), lambda g: (g, 0)),
        out_shape=jax.ShapeDtypeStruct((BP, H), jnp.float32),
    )(x, m, W1, b1r, W2, b2r)
    return out.reshape(B, P, H)
